# baseline (speedup 1.0000x reference)
import functools

import jax
import jax.numpy as jnp
from jax import lax
from jax.experimental import pallas as pl
from jax.experimental.pallas import tpu as pltpu
from jax.experimental.pallas import tpu_sc as plsc

BATCH = 16384
BINS = 50
DIM = 16

NUM_CORES = 2
NUM_SUBCORES = 16
NUM_WORKERS = NUM_CORES * NUM_SUBCORES

ROWS_PER_WORKER = BATCH // NUM_WORKERS
CHUNK = 32
NCHUNKS = ROWS_PER_WORKER // CHUNK

_mesh = plsc.VectorSubcoreMesh(core_axis_name="c", subcore_axis_name="s")


@functools.partial(
    pl.kernel,
    mesh=_mesh,
    compiler_params=pltpu.CompilerParams(use_tc_tiling_on_sc=False),
    out_type=jax.ShapeDtypeStruct((BATCH, DIM), jnp.float32),
    scratch_types=[
        pltpu.VMEM((ROWS_PER_WORKER, 56), jnp.int32),
        pltpu.VMEM((CHUNK * 56, DIM), jnp.float32),
        pltpu.VMEM((CHUNK * 56, DIM), jnp.float32),
        pltpu.VMEM((CHUNK, DIM), jnp.float32),
        pltpu.SemaphoreType.DMA,
        pltpu.SemaphoreType.DMA,
    ],
)
def _pooled_lookup(
    table_hbm, idx_hbm, out_hbm, idx_v, rows_a, rows_b, out_v, sem_a, sem_b
):
    wid = lax.axis_index("s") * NUM_CORES + lax.axis_index("c")
    out_base = wid * ROWS_PER_WORKER

    pltpu.sync_copy(
        idx_hbm.at[pl.ds(out_base, ROWS_PER_WORKER), pl.ds(0, 56)], idx_v
    )

    bufs = (rows_a, rows_b)
    sems = (sem_a, sem_b)

    def fire(g):
        buf, sem = bufs[g % 2], sems[g % 2]
        return [
            pltpu.async_copy(
                table_hbm.at[idx_v.at[g * CHUNK + i]],
                buf.at[pl.ds(i * 56, 56)],
                sem,
            )
            for i in range(CHUNK)
        ]

    def accumulate(g):
        buf = bufs[g % 2]

        def acc_body(i, carry):
            r = i * 56
            a0 = buf[r, :]
            a1 = buf[r + 1, :]
            a2 = buf[r + 2, :]
            a3 = buf[r + 3, :]
            for j in range(4, BINS - 2, 4):
                a0 = a0 + buf[r + j, :]
                a1 = a1 + buf[r + j + 1, :]
                a2 = a2 + buf[r + j + 2, :]
                a3 = a3 + buf[r + j + 3, :]
            a0 = a0 + buf[r + BINS - 2, :]
            a1 = a1 + buf[r + BINS - 1, :]
            out_v[i, :] = ((a0 + a1) + (a2 + a3)) * jnp.float32(1.0 / BINS)
            return carry

        lax.fori_loop(0, CHUNK, acc_body, 0)

    pending = fire(0)
    for g in range(NCHUNKS):
        nxt = fire(g + 1) if g + 1 < NCHUNKS else None
        for c in pending:
            c.wait()
        accumulate(g)
        pltpu.sync_copy(out_v, out_hbm.at[pl.ds(out_base + g * CHUNK, CHUNK)])
        pending = nxt


def kernel(bin_indices, embedding_weight):
    idx_pad = jnp.pad(bin_indices.astype(jnp.int32), ((0, 0), (0, 128 - BINS)))
    return _pooled_lookup(embedding_weight, idx_pad)


# speedup vs baseline: 5.0426x; 5.0426x over previous
import functools

import jax
import jax.numpy as jnp
from jax import lax
from jax.experimental import pallas as pl
from jax.experimental.pallas import tpu as pltpu
from jax.experimental.pallas import tpu_sc as plsc

BATCH = 16384
BINS = 50
BINS_PAD = 64
DIM = 16

NUM_CORES = 2
NUM_SUBCORES = 16
NUM_WORKERS = NUM_CORES * NUM_SUBCORES

ROWS_PER_WORKER = BATCH // NUM_WORKERS
QUARTER = 128
NQ = ROWS_PER_WORKER // QUARTER

_mesh = plsc.VectorSubcoreMesh(core_axis_name="c", subcore_axis_name="s")


@functools.partial(
    pl.kernel,
    mesh=_mesh,
    compiler_params=pltpu.CompilerParams(use_tc_tiling_on_sc=False),
    out_type=jax.ShapeDtypeStruct((BATCH, DIM), jnp.float32),
    scratch_types=[
        pltpu.VMEM((BINS_PAD, ROWS_PER_WORKER), jnp.int32),
        pltpu.VMEM((ROWS_PER_WORKER, DIM), jnp.float32),
        pltpu.SemaphoreType.DMA,
    ],
)
def _pooled_lookup(table_hbm, idxt_hbm, out_hbm, idx_v, acc_v, sem):
    wid = lax.axis_index("s") * NUM_CORES + lax.axis_index("c")
    out_base = wid * ROWS_PER_WORKER

    pltpu.sync_copy(idxt_hbm.at[:, pl.ds(out_base, ROWS_PER_WORKER)], idx_v)

    def zero_body(i, carry):
        acc_v[i, :] = jnp.zeros((DIM,), jnp.float32)
        return carry

    lax.fori_loop(0, ROWS_PER_WORKER, zero_body, 0)

    copies = [
        pltpu.async_copy(
            table_hbm.at[idx_v.at[j, pl.ds(q * QUARTER, QUARTER)]],
            acc_v.at[pl.ds(q * QUARTER, QUARTER)],
            sem,
            add=True,
        )
        for q in range(NQ)
        for j in range(BINS)
    ]
    for c in copies:
        c.wait()

    def scale_body(i, carry):
        acc_v[i, :] = acc_v[i, :] * jnp.float32(1.0 / BINS)
        return carry

    lax.fori_loop(0, ROWS_PER_WORKER, scale_body, 0)
    pltpu.sync_copy(acc_v, out_hbm.at[pl.ds(out_base, ROWS_PER_WORKER)])


def kernel(bin_indices, embedding_weight):
    idxt = jnp.pad(
        bin_indices.astype(jnp.int32), ((0, 0), (0, BINS_PAD - BINS))
    ).T
    return _pooled_lookup(embedding_weight, idxt)
